# baseline (speedup 1.0000x reference)
import functools

import jax
import jax.numpy as jnp
from jax import lax
from jax.experimental import pallas as pl
from jax.experimental.pallas import tpu as pltpu
from jax.experimental.pallas import tpu_sc as plsc

N = 8192
M = 8192
NUM_CORES = 2
NUM_SUBCORES = 16
NUM_WORKERS = NUM_CORES * NUM_SUBCORES
SC_ROWS = 3584
TC_ROWS = N - SC_ROWS
TC_BLOCK = 512
ROWS_PER_WORKER = SC_ROWS // NUM_WORKERS
LANES = 16
CHUNKS_PER_JBLOCK = 8
JBLOCK = CHUNKS_PER_JBLOCK * LANES
NUM_JBLOCKS = M // JBLOCK

_INF = float("inf")


def _sc_body(bx_hbm, by_hbm, ra_hbm, rb_hbm, rowmin_hbm, colpart_hbm,
             bx_v, by_v, ra_v, rb_v, colmin_v, rowmin_v, racc_v):
  wid = lax.axis_index("s") * NUM_CORES + lax.axis_index("c")
  base = wid * ROWS_PER_WORKER

  pltpu.sync_copy(bx_hbm.at[pl.ds(base, ROWS_PER_WORKER)], bx_v)
  pltpu.sync_copy(by_hbm.at[pl.ds(base, ROWS_PER_WORKER)], by_v)
  pltpu.sync_copy(ra_hbm, ra_v)
  pltpu.sync_copy(rb_hbm, rb_v)

  inf_vec = jnp.full((LANES,), _INF, dtype=jnp.float32)

  def init_body(i, _):
    colmin_v[pl.ds(i * LANES, LANES)] = inf_vec
    return 0

  lax.fori_loop(0, M // LANES, init_body, 0)

  lane_id = lax.iota(jnp.int32, LANES)
  gather_dnums = lax.GatherDimensionNumbers(
      offset_dims=(), collapsed_slice_dims=(0,), start_index_map=(0,))

  def lane_min_all(v):
    for s in (8, 4, 2, 1):
      idx = ((lane_id + s) & (LANES - 1)).reshape(LANES, 1)
      rot = lax.gather(v, idx, gather_dnums, (1,),
                       mode=lax.GatherScatterMode.PROMISE_IN_BOUNDS)
      v = jnp.minimum(v, rot)
    return v

  def g_body(g, _):
    bxg = bx_v[pl.ds(g * LANES, LANES)]
    byg = by_v[pl.ds(g * LANES, LANES)]
    for k in range(LANES):
      racc_v[pl.ds(k * LANES, LANES)] = inf_vec

    def jb_body(jb, _):
      j0 = jb * JBLOCK
      ras = [ra_v[pl.ds(j0 + c * LANES, LANES)]
             for c in range(CHUNKS_PER_JBLOCK)]
      rbs = [rb_v[pl.ds(j0 + c * LANES, LANES)]
             for c in range(CHUNKS_PER_JBLOCK)]
      cols = [colmin_v[pl.ds(j0 + c * LANES, LANES)]
              for c in range(CHUNKS_PER_JBLOCK)]
      for k in range(LANES):
        bxs = jnp.full((LANES,), bxg[k], dtype=jnp.float32)
        bys = jnp.full((LANES,), byg[k], dtype=jnp.float32)
        racc = racc_v[pl.ds(k * LANES, LANES)]
        for c in range(CHUNKS_PER_JBLOCK):
          dx = bxs - ras[c]
          dy = bys - rbs[c]
          d2 = dx * dx + dy * dy
          cols[c] = jnp.minimum(cols[c], d2)
          racc = jnp.minimum(racc, d2)
        racc_v[pl.ds(k * LANES, LANES)] = racc
      for c in range(CHUNKS_PER_JBLOCK):
        colmin_v[pl.ds(j0 + c * LANES, LANES)] = cols[c]
      return 0

    lax.fori_loop(0, NUM_JBLOCKS, jb_body, 0)

    packed = inf_vec
    for k in range(LANES):
      m = lane_min_all(racc_v[pl.ds(k * LANES, LANES)])
      packed = jnp.where(lane_id == k, m, packed)
    rowmin_v[pl.ds(g * LANES, LANES)] = packed
    return 0

  lax.fori_loop(0, ROWS_PER_WORKER // LANES, g_body, 0)

  pltpu.sync_copy(rowmin_v, rowmin_hbm.at[pl.ds(base, ROWS_PER_WORKER)])
  pltpu.sync_copy(colmin_v, colpart_hbm.at[wid])


_sc_pairwise_min = functools.partial(
    pl.kernel,
    out_type=(
        jax.ShapeDtypeStruct((SC_ROWS,), jnp.float32),
        jax.ShapeDtypeStruct((NUM_WORKERS, M), jnp.float32),
    ),
    mesh=plsc.VectorSubcoreMesh(
        core_axis_name="c", subcore_axis_name="s",
        num_cores=NUM_CORES, num_subcores=NUM_SUBCORES),
    scratch_types=[
        pltpu.VMEM((ROWS_PER_WORKER,), jnp.float32),
        pltpu.VMEM((ROWS_PER_WORKER,), jnp.float32),
        pltpu.VMEM((M,), jnp.float32),
        pltpu.VMEM((M,), jnp.float32),
        pltpu.VMEM((M,), jnp.float32),
        pltpu.VMEM((ROWS_PER_WORKER,), jnp.float32),
        pltpu.VMEM((LANES * LANES,), jnp.float32),
    ],
)(_sc_body)


def _tc_rows_body(bez_ref, ra_ref, rb_ref, rowmin_ref, colpart_ref,
                  acc_ref):
  i = pl.program_id(0)
  bx_col = bez_ref[:, 0:1]
  by_col = bez_ref[:, 1:2]
  dx = bx_col - ra_ref[...]
  dy = by_col - rb_ref[...]
  d2 = dx * dx + dy * dy
  rowmin_ref[...] = jnp.min(d2, axis=1, keepdims=True)
  colc = jnp.min(d2, axis=0, keepdims=True)

  @pl.when(i == 0)
  def _():
    acc_ref[...] = jnp.full((1, M), _INF, dtype=jnp.float32)

  acc_ref[...] = jnp.minimum(acc_ref[...], colc)
  colpart_ref[...] = acc_ref[...]


_tc_pairwise_min = pl.pallas_call(
    _tc_rows_body,
    grid=(TC_ROWS // TC_BLOCK,),
    in_specs=[
        pl.BlockSpec((TC_BLOCK, 2), lambda i: (i + SC_ROWS // TC_BLOCK, 0)),
        pl.BlockSpec((1, M), lambda i: (0, 0)),
        pl.BlockSpec((1, M), lambda i: (0, 0)),
    ],
    out_specs=[
        pl.BlockSpec((TC_BLOCK, 1), lambda i: (i, 0)),
        pl.BlockSpec((1, M), lambda i: (0, 0)),
    ],
    out_shape=[
        jax.ShapeDtypeStruct((TC_ROWS, 1), jnp.float32),
        jax.ShapeDtypeStruct((1, M), jnp.float32),
    ],
    scratch_shapes=[pltpu.VMEM((1, M), jnp.float32)],
)


def _finish_body(rm_sc_ref, rm_tc_ref, colpart_sc_ref, colpart_tc_ref,
                 bx_ref, by_ref, out_ref):
  zero = jnp.float32(0.0)
  scr = SC_ROWS // 128
  rowd_sc = jnp.sqrt(jnp.maximum(rm_sc_ref[...], zero))
  rowd_tc = jnp.sqrt(jnp.maximum(rm_tc_ref[...], zero))
  bx = bx_ref[...]
  by = by_ref[...]
  bound = jnp.float32(2000.0)
  mask = ((bx >= -bound) & (bx <= bound) &
          (by >= -bound) & (by <= bound))
  maskf = mask.astype(jnp.float32)
  n_kept = jnp.maximum(jnp.sum(maskf), jnp.float32(1.0))
  sum1 = (jnp.sum(jnp.where(mask[:scr], rowd_sc, zero)) +
          jnp.sum(jnp.where(mask[scr:], rowd_tc, zero)))
  mean1 = sum1 / n_kept

  colmin = jnp.minimum(
      jnp.min(colpart_sc_ref[...], axis=0, keepdims=True),
      colpart_tc_ref[...])
  mean2 = jnp.sum(jnp.sqrt(jnp.maximum(colmin, zero))) / jnp.float32(M)

  out_ref[...] = ((mean1 + mean2) * jnp.float32(0.5)).reshape(1, 1)


def kernel(bezier_proj_centerline_img, ref_catheter_centerline):
  bez = bezier_proj_centerline_img
  ref = ref_catheter_centerline
  bx = bez[:, 0]
  by = bez[:, 1]
  ra = ref[:, 1]
  rb = ref[:, 0]

  rowmin2_sc, colpart2_sc = _sc_pairwise_min(bx, by, ra, rb)
  rowmin2_tc, colpart2_tc = _tc_pairwise_min(
      bez, ra.reshape(1, M), rb.reshape(1, M))

  out = pl.pallas_call(
      _finish_body,
      out_shape=jax.ShapeDtypeStruct((1, 1), jnp.float32),
  )(rowmin2_sc.reshape(SC_ROWS // 128, 128),
    rowmin2_tc.reshape(TC_ROWS // 128, 128),
    colpart2_sc, colpart2_tc,
    bx.reshape(64, 128), by.reshape(64, 128))
  return out[0, 0]


# speedup vs baseline: 1.1151x; 1.1151x over previous
import functools

import jax
import jax.numpy as jnp
from jax import lax
from jax.experimental import pallas as pl
from jax.experimental.pallas import tpu as pltpu
from jax.experimental.pallas import tpu_sc as plsc

N = 8192
M = 8192
NUM_CORES = 2
NUM_SUBCORES = 16
NUM_WORKERS = NUM_CORES * NUM_SUBCORES
SC_ROWS = 3072
TC_ROWS = N - SC_ROWS
TC_BLOCK = 512
ROWS_PER_WORKER = SC_ROWS // NUM_WORKERS
LANES = 16
CHUNKS_PER_JBLOCK = 8
JBLOCK = CHUNKS_PER_JBLOCK * LANES
NUM_JBLOCKS = M // JBLOCK

_INF = float("inf")


def _sc_body(bx_hbm, by_hbm, ra_hbm, rb_hbm, rowmin_hbm, colpart_hbm,
             bx_v, by_v, ra_v, rb_v, colmin_v, rowmin_v, racc_v):
  wid = lax.axis_index("s") * NUM_CORES + lax.axis_index("c")
  base = wid * ROWS_PER_WORKER

  pltpu.sync_copy(bx_hbm.at[pl.ds(base, ROWS_PER_WORKER)], bx_v)
  pltpu.sync_copy(by_hbm.at[pl.ds(base, ROWS_PER_WORKER)], by_v)
  pltpu.sync_copy(ra_hbm, ra_v)
  pltpu.sync_copy(rb_hbm, rb_v)

  inf_vec = jnp.full((LANES,), _INF, dtype=jnp.float32)

  def init_body(i, _):
    colmin_v[pl.ds(i * LANES, LANES)] = inf_vec
    return 0

  lax.fori_loop(0, M // LANES, init_body, 0)

  lane_id = lax.iota(jnp.int32, LANES)
  gather_dnums = lax.GatherDimensionNumbers(
      offset_dims=(), collapsed_slice_dims=(0,), start_index_map=(0,))

  def lane_min_all(v):
    for s in (8, 4, 2, 1):
      idx = ((lane_id + s) & (LANES - 1)).reshape(LANES, 1)
      rot = lax.gather(v, idx, gather_dnums, (1,),
                       mode=lax.GatherScatterMode.PROMISE_IN_BOUNDS)
      v = jnp.minimum(v, rot)
    return v

  def g_body(g, _):
    bxg = bx_v[pl.ds(g * LANES, LANES)]
    byg = by_v[pl.ds(g * LANES, LANES)]
    for k in range(LANES):
      racc_v[pl.ds(k * LANES, LANES)] = inf_vec

    def jb_body(jb, _):
      j0 = jb * JBLOCK
      ras = [ra_v[pl.ds(j0 + c * LANES, LANES)]
             for c in range(CHUNKS_PER_JBLOCK)]
      rbs = [rb_v[pl.ds(j0 + c * LANES, LANES)]
             for c in range(CHUNKS_PER_JBLOCK)]
      cols = [colmin_v[pl.ds(j0 + c * LANES, LANES)]
              for c in range(CHUNKS_PER_JBLOCK)]
      for k in range(LANES):
        bxs = jnp.full((LANES,), bxg[k], dtype=jnp.float32)
        bys = jnp.full((LANES,), byg[k], dtype=jnp.float32)
        racc = racc_v[pl.ds(k * LANES, LANES)]
        for c in range(CHUNKS_PER_JBLOCK):
          dx = bxs - ras[c]
          dy = bys - rbs[c]
          d2 = dx * dx + dy * dy
          cols[c] = jnp.minimum(cols[c], d2)
          racc = jnp.minimum(racc, d2)
        racc_v[pl.ds(k * LANES, LANES)] = racc
      for c in range(CHUNKS_PER_JBLOCK):
        colmin_v[pl.ds(j0 + c * LANES, LANES)] = cols[c]
      return 0

    lax.fori_loop(0, NUM_JBLOCKS, jb_body, 0)

    packed = inf_vec
    for k in range(LANES):
      m = lane_min_all(racc_v[pl.ds(k * LANES, LANES)])
      packed = jnp.where(lane_id == k, m, packed)
    rowmin_v[pl.ds(g * LANES, LANES)] = packed
    return 0

  lax.fori_loop(0, ROWS_PER_WORKER // LANES, g_body, 0)

  pltpu.sync_copy(rowmin_v, rowmin_hbm.at[pl.ds(base, ROWS_PER_WORKER)])
  pltpu.sync_copy(colmin_v, colpart_hbm.at[wid])


_sc_pairwise_min = functools.partial(
    pl.kernel,
    out_type=(
        jax.ShapeDtypeStruct((SC_ROWS,), jnp.float32),
        jax.ShapeDtypeStruct((NUM_WORKERS, M), jnp.float32),
    ),
    mesh=plsc.VectorSubcoreMesh(
        core_axis_name="c", subcore_axis_name="s",
        num_cores=NUM_CORES, num_subcores=NUM_SUBCORES),
    scratch_types=[
        pltpu.VMEM((ROWS_PER_WORKER,), jnp.float32),
        pltpu.VMEM((ROWS_PER_WORKER,), jnp.float32),
        pltpu.VMEM((M,), jnp.float32),
        pltpu.VMEM((M,), jnp.float32),
        pltpu.VMEM((M,), jnp.float32),
        pltpu.VMEM((ROWS_PER_WORKER,), jnp.float32),
        pltpu.VMEM((LANES * LANES,), jnp.float32),
    ],
)(_sc_body)


def _tc_rows_body(bez_ref, ra_ref, rb_ref, rowmin_ref, colpart_ref,
                  acc_ref):
  i = pl.program_id(0)
  bx_col = bez_ref[:, 0:1]
  by_col = bez_ref[:, 1:2]
  ch = 1024
  bxb = jnp.broadcast_to(bx_col, (TC_BLOCK, ch))
  byb = jnp.broadcast_to(by_col, (TC_BLOCK, ch))
  racc = jnp.full((TC_BLOCK, ch), _INF, dtype=jnp.float32)
  colcs = []
  for c in range(M // ch):
    rch = ra_ref[:, c * ch:(c + 1) * ch]
    rbh = rb_ref[:, c * ch:(c + 1) * ch]
    dx = bxb - rch
    dy = byb - rbh
    d2 = dx * dx + dy * dy
    racc = jnp.minimum(racc, d2)
    colcs.append(jnp.min(d2, axis=0, keepdims=True))
  rowmin_ref[...] = jnp.min(racc, axis=1, keepdims=True)
  colc = jnp.concatenate(colcs, axis=1)

  @pl.when(i == 0)
  def _():
    acc_ref[...] = jnp.full((1, M), _INF, dtype=jnp.float32)

  acc_ref[...] = jnp.minimum(acc_ref[...], colc)
  colpart_ref[...] = acc_ref[...]


_tc_pairwise_min = pl.pallas_call(
    _tc_rows_body,
    grid=(TC_ROWS // TC_BLOCK,),
    in_specs=[
        pl.BlockSpec((TC_BLOCK, 2), lambda i: (i + SC_ROWS // TC_BLOCK, 0)),
        pl.BlockSpec((1, M), lambda i: (0, 0)),
        pl.BlockSpec((1, M), lambda i: (0, 0)),
    ],
    out_specs=[
        pl.BlockSpec((TC_BLOCK, 1), lambda i: (i, 0)),
        pl.BlockSpec((1, M), lambda i: (0, 0)),
    ],
    out_shape=[
        jax.ShapeDtypeStruct((TC_ROWS, 1), jnp.float32),
        jax.ShapeDtypeStruct((1, M), jnp.float32),
    ],
    scratch_shapes=[pltpu.VMEM((1, M), jnp.float32)],
)


def _finish_body(rm_sc_ref, rm_tc_ref, colpart_sc_ref, colpart_tc_ref,
                 bx_ref, by_ref, out_ref):
  zero = jnp.float32(0.0)
  scr = SC_ROWS // 128
  rowd_sc = jnp.sqrt(jnp.maximum(rm_sc_ref[...], zero))
  rowd_tc = jnp.sqrt(jnp.maximum(rm_tc_ref[...], zero))
  bx = bx_ref[...]
  by = by_ref[...]
  bound = jnp.float32(2000.0)
  mask = ((bx >= -bound) & (bx <= bound) &
          (by >= -bound) & (by <= bound))
  maskf = mask.astype(jnp.float32)
  n_kept = jnp.maximum(jnp.sum(maskf), jnp.float32(1.0))
  sum1 = (jnp.sum(jnp.where(mask[:scr], rowd_sc, zero)) +
          jnp.sum(jnp.where(mask[scr:], rowd_tc, zero)))
  mean1 = sum1 / n_kept

  colmin = jnp.minimum(
      jnp.min(colpart_sc_ref[...], axis=0, keepdims=True),
      colpart_tc_ref[...])
  mean2 = jnp.sum(jnp.sqrt(jnp.maximum(colmin, zero))) / jnp.float32(M)

  out_ref[...] = ((mean1 + mean2) * jnp.float32(0.5)).reshape(1, 1)


def kernel(bezier_proj_centerline_img, ref_catheter_centerline):
  bez = bezier_proj_centerline_img
  ref = ref_catheter_centerline
  bx = bez[:, 0]
  by = bez[:, 1]
  ra = ref[:, 1]
  rb = ref[:, 0]

  rowmin2_sc, colpart2_sc = _sc_pairwise_min(bx, by, ra, rb)
  rowmin2_tc, colpart2_tc = _tc_pairwise_min(
      bez, ra.reshape(1, M), rb.reshape(1, M))

  out = pl.pallas_call(
      _finish_body,
      out_shape=jax.ShapeDtypeStruct((1, 1), jnp.float32),
  )(rowmin2_sc.reshape(SC_ROWS // 128, 128),
    rowmin2_tc.reshape(TC_ROWS // 128, 128),
    colpart2_sc, colpart2_tc,
    bx.reshape(64, 128), by.reshape(64, 128))
  return out[0, 0]


# speedup vs baseline: 1.2593x; 1.1293x over previous
import functools

import jax
import jax.numpy as jnp
from jax import lax
from jax.experimental import pallas as pl
from jax.experimental.pallas import tpu as pltpu
from jax.experimental.pallas import tpu_sc as plsc

N = 8192
M = 8192
NUM_CORES = 2
NUM_SUBCORES = 16
NUM_WORKERS = NUM_CORES * NUM_SUBCORES
SC_ROWS = 2560
TC_ROWS = N - SC_ROWS
TC_BLOCK = 512
ROWS_PER_WORKER = SC_ROWS // NUM_WORKERS
LANES = 16
CHUNKS_PER_JBLOCK = 8
JBLOCK = CHUNKS_PER_JBLOCK * LANES
NUM_JBLOCKS = M // JBLOCK

_INF = float("inf")


def _sc_body(bx_hbm, by_hbm, ra_hbm, rb_hbm, rowmin_hbm, colpart_hbm,
             bx_v, by_v, ra_v, rb_v, colmin_v, rowmin_v, racc_v):
  wid = lax.axis_index("s") * NUM_CORES + lax.axis_index("c")
  base = wid * ROWS_PER_WORKER

  pltpu.sync_copy(bx_hbm.at[pl.ds(base, ROWS_PER_WORKER)], bx_v)
  pltpu.sync_copy(by_hbm.at[pl.ds(base, ROWS_PER_WORKER)], by_v)
  pltpu.sync_copy(ra_hbm, ra_v)
  pltpu.sync_copy(rb_hbm, rb_v)

  inf_vec = jnp.full((LANES,), _INF, dtype=jnp.float32)

  def init_body(i, _):
    colmin_v[pl.ds(i * LANES, LANES)] = inf_vec
    return 0

  lax.fori_loop(0, M // LANES, init_body, 0)

  lane_id = lax.iota(jnp.int32, LANES)
  gather_dnums = lax.GatherDimensionNumbers(
      offset_dims=(), collapsed_slice_dims=(0,), start_index_map=(0,))

  def lane_min_all(v):
    for s in (8, 4, 2, 1):
      idx = ((lane_id + s) & (LANES - 1)).reshape(LANES, 1)
      rot = lax.gather(v, idx, gather_dnums, (1,),
                       mode=lax.GatherScatterMode.PROMISE_IN_BOUNDS)
      v = jnp.minimum(v, rot)
    return v

  def g_body(g, _):
    bxg = bx_v[pl.ds(g * LANES, LANES)]
    byg = by_v[pl.ds(g * LANES, LANES)]
    for k in range(LANES):
      racc_v[pl.ds(k * LANES, LANES)] = inf_vec

    def jb_body(jb, _):
      j0 = jb * JBLOCK
      ras = [ra_v[pl.ds(j0 + c * LANES, LANES)]
             for c in range(CHUNKS_PER_JBLOCK)]
      rbs = [rb_v[pl.ds(j0 + c * LANES, LANES)]
             for c in range(CHUNKS_PER_JBLOCK)]
      cols = [colmin_v[pl.ds(j0 + c * LANES, LANES)]
              for c in range(CHUNKS_PER_JBLOCK)]
      for k in range(LANES):
        bxs = jnp.full((LANES,), bxg[k], dtype=jnp.float32)
        bys = jnp.full((LANES,), byg[k], dtype=jnp.float32)
        racc = racc_v[pl.ds(k * LANES, LANES)]
        for c in range(CHUNKS_PER_JBLOCK):
          dx = bxs - ras[c]
          dy = bys - rbs[c]
          d2 = dx * dx + dy * dy
          cols[c] = jnp.minimum(cols[c], d2)
          racc = jnp.minimum(racc, d2)
        racc_v[pl.ds(k * LANES, LANES)] = racc
      for c in range(CHUNKS_PER_JBLOCK):
        colmin_v[pl.ds(j0 + c * LANES, LANES)] = cols[c]
      return 0

    lax.fori_loop(0, NUM_JBLOCKS, jb_body, 0)

    packed = inf_vec
    for k in range(LANES):
      m = lane_min_all(racc_v[pl.ds(k * LANES, LANES)])
      packed = jnp.where(lane_id == k, m, packed)
    rowmin_v[pl.ds(g * LANES, LANES)] = packed
    return 0

  lax.fori_loop(0, ROWS_PER_WORKER // LANES, g_body, 0)

  pltpu.sync_copy(rowmin_v, rowmin_hbm.at[pl.ds(base, ROWS_PER_WORKER)])
  pltpu.sync_copy(colmin_v, colpart_hbm.at[wid])


_sc_pairwise_min = functools.partial(
    pl.kernel,
    out_type=(
        jax.ShapeDtypeStruct((SC_ROWS,), jnp.float32),
        jax.ShapeDtypeStruct((NUM_WORKERS, M), jnp.float32),
    ),
    mesh=plsc.VectorSubcoreMesh(
        core_axis_name="c", subcore_axis_name="s",
        num_cores=NUM_CORES, num_subcores=NUM_SUBCORES),
    scratch_types=[
        pltpu.VMEM((ROWS_PER_WORKER,), jnp.float32),
        pltpu.VMEM((ROWS_PER_WORKER,), jnp.float32),
        pltpu.VMEM((M,), jnp.float32),
        pltpu.VMEM((M,), jnp.float32),
        pltpu.VMEM((M,), jnp.float32),
        pltpu.VMEM((ROWS_PER_WORKER,), jnp.float32),
        pltpu.VMEM((LANES * LANES,), jnp.float32),
    ],
)(_sc_body)


def _tc_rows_body(bez_ref, ra_ref, rb_ref, rowmin_ref, colpart_ref,
                  acc_ref):
  i = pl.program_id(0)
  bx_col = bez_ref[:, 0:1]
  by_col = bez_ref[:, 1:2]
  ch = 1024
  bxb = jnp.broadcast_to(bx_col, (TC_BLOCK, ch))
  byb = jnp.broadcast_to(by_col, (TC_BLOCK, ch))
  racc = jnp.full((TC_BLOCK, ch), _INF, dtype=jnp.float32)
  colcs = []
  for c in range(M // ch):
    rch = ra_ref[:, c * ch:(c + 1) * ch]
    rbh = rb_ref[:, c * ch:(c + 1) * ch]
    dx = bxb - rch
    dy = byb - rbh
    d2 = dx * dx + dy * dy
    racc = jnp.minimum(racc, d2)
    colcs.append(jnp.min(d2, axis=0, keepdims=True))
  rowmin_ref[...] = jnp.min(racc, axis=1, keepdims=True)
  colc = jnp.concatenate(colcs, axis=1)

  @pl.when(i == 0)
  def _():
    acc_ref[...] = jnp.full((1, M), _INF, dtype=jnp.float32)

  acc_ref[...] = jnp.minimum(acc_ref[...], colc)
  colpart_ref[...] = acc_ref[...]


_tc_pairwise_min = pl.pallas_call(
    _tc_rows_body,
    grid=(TC_ROWS // TC_BLOCK,),
    in_specs=[
        pl.BlockSpec((TC_BLOCK, 2), lambda i: (i + SC_ROWS // TC_BLOCK, 0)),
        pl.BlockSpec((1, M), lambda i: (0, 0)),
        pl.BlockSpec((1, M), lambda i: (0, 0)),
    ],
    out_specs=[
        pl.BlockSpec((TC_BLOCK, 1), lambda i: (i, 0)),
        pl.BlockSpec((1, M), lambda i: (0, 0)),
    ],
    out_shape=[
        jax.ShapeDtypeStruct((TC_ROWS, 1), jnp.float32),
        jax.ShapeDtypeStruct((1, M), jnp.float32),
    ],
    scratch_shapes=[pltpu.VMEM((1, M), jnp.float32)],
)


def _finish_body(rm_sc_ref, rm_tc_ref, colpart_sc_ref, colpart_tc_ref,
                 bx_ref, by_ref, out_ref):
  zero = jnp.float32(0.0)
  scr = SC_ROWS // 128
  rowd_sc = jnp.sqrt(jnp.maximum(rm_sc_ref[...], zero))
  rowd_tc = jnp.sqrt(jnp.maximum(rm_tc_ref[...], zero))
  bx = bx_ref[...]
  by = by_ref[...]
  bound = jnp.float32(2000.0)
  mask = ((bx >= -bound) & (bx <= bound) &
          (by >= -bound) & (by <= bound))
  maskf = mask.astype(jnp.float32)
  n_kept = jnp.maximum(jnp.sum(maskf), jnp.float32(1.0))
  sum1 = (jnp.sum(jnp.where(mask[:scr], rowd_sc, zero)) +
          jnp.sum(jnp.where(mask[scr:], rowd_tc, zero)))
  mean1 = sum1 / n_kept

  colmin = jnp.minimum(
      jnp.min(colpart_sc_ref[...], axis=0, keepdims=True),
      colpart_tc_ref[...])
  mean2 = jnp.sum(jnp.sqrt(jnp.maximum(colmin, zero))) / jnp.float32(M)

  out_ref[...] = ((mean1 + mean2) * jnp.float32(0.5)).reshape(1, 1)


def kernel(bezier_proj_centerline_img, ref_catheter_centerline):
  bez = bezier_proj_centerline_img
  ref = ref_catheter_centerline
  bx = bez[:, 0]
  by = bez[:, 1]
  ra = ref[:, 1]
  rb = ref[:, 0]

  rowmin2_sc, colpart2_sc = _sc_pairwise_min(bx, by, ra, rb)
  rowmin2_tc, colpart2_tc = _tc_pairwise_min(
      bez, ra.reshape(1, M), rb.reshape(1, M))

  out = pl.pallas_call(
      _finish_body,
      out_shape=jax.ShapeDtypeStruct((1, 1), jnp.float32),
  )(rowmin2_sc.reshape(SC_ROWS // 128, 128),
    rowmin2_tc.reshape(TC_ROWS // 128, 128),
    colpart2_sc, colpart2_tc,
    bx.reshape(64, 128), by.reshape(64, 128))
  return out[0, 0]
